# baseline (speedup 1.0000x reference)
import dataclasses
import functools

import jax
import jax.numpy as jnp
from jax import lax
from jax.experimental import pallas as pl
from jax.experimental.pallas import tpu as pltpu
from jax.experimental.pallas import tpu_sc as plsc

N = 10000
E = 320000
D = 128

NC = 2
NS = 16
NW = NC * NS
EPW = E // NW
NBUF = 4
CH = 25
NCHUNK = EPW // CH
assert NCHUNK % NBUF == 0
NROW = 640
RBASE = 624
RSPAN = 640

_MESH = plsc.VectorSubcoreMesh(core_axis_name="c", subcore_axis_name="s")

_SC_PARAMS = pltpu.CompilerParams()
if "needs_layout_passes" in pltpu.CompilerParams.__dataclass_fields__:
    _SC_PARAMS = dataclasses.replace(_SC_PARAMS, needs_layout_passes=False)
_SC_FLAT = dataclasses.replace(_SC_PARAMS, use_tc_tiling_on_sc=False)


@functools.partial(
    pl.kernel,
    out_type=jax.ShapeDtypeStruct((NC, NROW, 16), jnp.float32),
    mesh=_MESH,
    scratch_types=[
        pltpu.VMEM((EPW // 16, 16), jnp.int32),
        pltpu.VMEM((NROW, 16), jnp.float32),
        pltpu.VMEM((NROW // 128, 128), jnp.int32),
        pltpu.VMEM_SHARED((NROW, 16), jnp.float32),
    ],
    compiler_params=_SC_FLAT,
)
def _deg_kernel(dst_hbm, out_hbm, dstbuf, partial, idrows, acc):
    c = lax.axis_index("c")
    s = lax.axis_index("s")
    wid = c * NS + s
    pltpu.sync_copy(dst_hbm.at[wid], dstbuf)

    zeros16 = jnp.zeros((16,), jnp.float32)

    @pl.loop(0, NROW)
    def _zero(r):
        partial[r, :] = zeros16

    iota16 = lax.broadcasted_iota(jnp.int32, (16,), 0)
    for k in range(NROW // 128):
        for j in range(8):
            idrows[k, pl.ds(j * 16, 16)] = iota16 + (k * 128 + j * 16)

    rows_per = NROW // NS
    pltpu.sync_copy(partial.at[pl.ds(s * rows_per, rows_per)],
                    acc.at[pl.ds(s * rows_per, rows_per)])
    plsc.subcore_barrier()

    @pl.loop(0, EPW // 16)
    def _count(i):
        n = dstbuf[i, :]
        row = jnp.right_shift(n, 4)
        col = jnp.bitwise_and(n, 15)
        cnt, last = plsc.scan_count(n)
        plsc.addupdate_scatter(partial, [row, col], cnt.astype(jnp.float32),
                               mask=last)

    for k in range(NROW // 128):
        pltpu.sync_copy(partial.at[pl.ds(k * 128, 128)],
                        acc.at[idrows.at[k]], add=True)
    plsc.subcore_barrier()

    @pl.when(s == 0)
    def _writeback():
        pltpu.sync_copy(acc, out_hbm.at[c])


@functools.partial(
    pl.kernel,
    out_type=jax.ShapeDtypeStruct((NC, N, D), jnp.float32),
    mesh=_MESH,
    scratch_types=(
        [
            pltpu.VMEM((NCHUNK, CH), jnp.int32),
            pltpu.VMEM((NCHUNK, CH), jnp.int32),
            pltpu.VMEM_SHARED((N, D), jnp.float32),
        ]
        + [pltpu.VMEM((CH, D), jnp.float32) for _ in range(NBUF)]
        + [pltpu.SemaphoreType.DMA for _ in range(NBUF)]
    ),
    compiler_params=_SC_FLAT,
)
def _agg_kernel(h_hbm, src_hbm, dst_hbm, out_hbm, sidx, didx, acc,
                *bufs_and_sems):
    rbufs = bufs_and_sems[:NBUF]
    gsems = bufs_and_sems[NBUF:]
    c = lax.axis_index("c")
    s = lax.axis_index("s")
    wid = c * NS + s
    pltpu.sync_copy(src_hbm.at[wid], sidx)
    pltpu.sync_copy(dst_hbm.at[wid], didx)

    for j in range(NBUF - 1):
        pltpu.async_copy(h_hbm.at[sidx.at[j]], rbufs[j], gsems[j])

    zeros16 = jnp.zeros((16,), jnp.float32)
    zb = rbufs[NBUF - 1]

    @pl.loop(0, CH)
    def _zero(r):
        for j in range(D // 16):
            zb[r, pl.ds(j * 16, 16)] = zeros16

    base = s * RBASE
    for k in range(RSPAN // CH):
        pltpu.sync_copy(zb, acc.at[pl.ds(base + k * CH, CH)])
    if RSPAN % CH:
        pltpu.sync_copy(zb.at[pl.ds(0, RSPAN % CH)],
                        acc.at[pl.ds(base + (RSPAN // CH) * CH, RSPAN % CH)])
    pltpu.async_copy(h_hbm.at[sidx.at[NBUF - 1]], zb, gsems[NBUF - 1])
    plsc.subcore_barrier()

    @pl.loop(0, NCHUNK // NBUF - 1)
    def _edges(k):
        i0 = k * NBUF
        for j in range(NBUF):
            pltpu.make_async_copy(h_hbm.at[sidx.at[i0 + j]],
                                  rbufs[j], gsems[j]).wait()
            pltpu.sync_copy(rbufs[j], acc.at[didx.at[i0 + j]], add=True)
            pltpu.async_copy(h_hbm.at[sidx.at[i0 + NBUF + j]], rbufs[j],
                             gsems[j])

    i0 = NCHUNK - NBUF
    for j in range(NBUF):
        pltpu.make_async_copy(h_hbm.at[sidx.at[i0 + j]],
                              rbufs[j], gsems[j]).wait()
        pltpu.sync_copy(rbufs[j], acc.at[didx.at[i0 + j]], add=True)

    plsc.subcore_barrier()
    pltpu.sync_copy(acc.at[pl.ds(base, RSPAN)],
                    out_hbm.at[c].at[pl.ds(base, RSPAN)])


def _mm_scale_body(x_ref, w_ref, deg_ref, hs_ref, dis_ref):
    dis = lax.rsqrt(deg_ref[...] + 1.0)
    dis_ref[...] = dis
    h = jnp.dot(x_ref[...], w_ref[...], preferred_element_type=jnp.float32)
    hs_ref[...] = h * dis


def _comb_body(p_ref, hs_ref, dis_ref, b_ref, g_ref, be_ref, w_ref,
               o_ref):
    t = (p_ref[0] + p_ref[1] + hs_ref[...]) * dis_ref[...] + b_ref[...]
    mean = jnp.mean(t, axis=0, keepdims=True)
    var = jnp.mean((t - mean) ** 2, axis=0, keepdims=True)
    bn = (t - mean) * lax.rsqrt(var + 1e-5) * g_ref[...] + be_ref[...]
    r = jnp.maximum(bn, 0.0)
    h2 = jnp.dot(r, w_ref[...], preferred_element_type=jnp.float32)
    o_ref[...] = h2 * dis_ref[...]


def _fin_body(p_ref, hs_ref, dis_ref, b_ref, o_ref):
    o_ref[...] = ((p_ref[0] + p_ref[1] + hs_ref[...]) * dis_ref[...]
                  + b_ref[...])


_f32 = jnp.float32
_mm_scale = pl.pallas_call(
    _mm_scale_body,
    out_shape=(jax.ShapeDtypeStruct((N, D), _f32),
               jax.ShapeDtypeStruct((N, 1), _f32)))
_comb = pl.pallas_call(_comb_body, out_shape=jax.ShapeDtypeStruct((N, D), _f32))
_fin = pl.pallas_call(_fin_body, out_shape=jax.ShapeDtypeStruct((N, D), _f32))


def kernel(x, edge_index, W1, b1, gamma1, beta1, W2, b2):
    src_c = edge_index[0].reshape(NW, NCHUNK, CH)
    dst_c = edge_index[1].reshape(NW, NCHUNK, CH)
    dst_w = edge_index[1].reshape(NW, EPW // 16, 16)

    degp = _deg_kernel(dst_w)
    deg2 = (degp[0] + degp[1]).reshape(-1)[:N].reshape(N, 1)

    h1s, dis = _mm_scale(x, W1, deg2)
    p = _agg_kernel(h1s, src_c, dst_c)
    h2s = _comb(p, h1s, dis, b1.reshape(1, D), gamma1.reshape(1, D),
                beta1.reshape(1, D), W2)
    p2 = _agg_kernel(h2s, src_c, dst_c)
    return _fin(p2, h2s, dis, b2.reshape(1, D))


# speedup vs baseline: 1.3019x; 1.3019x over previous
import dataclasses
import functools

import jax
import jax.numpy as jnp
from jax import lax
from jax.experimental import pallas as pl
from jax.experimental.pallas import tpu as pltpu
from jax.experimental.pallas import tpu_sc as plsc

N = 10000
E = 320000
D = 128

NC = 2
NS = 16
NW = NC * NS
EPW = E // NW
NBUF = 8
CH = 24
NCHUNK = EPW // CH
TAIL = EPW - NCHUNK * CH
assert NCHUNK % NBUF == 0
NROW = 640
RBASE = 624
RSPAN = 640

_MESH = plsc.VectorSubcoreMesh(core_axis_name="c", subcore_axis_name="s")

_SC_PARAMS = pltpu.CompilerParams()
if "needs_layout_passes" in pltpu.CompilerParams.__dataclass_fields__:
    _SC_PARAMS = dataclasses.replace(_SC_PARAMS, needs_layout_passes=False)
_SC_FLAT = dataclasses.replace(_SC_PARAMS, use_tc_tiling_on_sc=False)


@functools.partial(
    pl.kernel,
    out_type=jax.ShapeDtypeStruct((NC, NROW, 16), jnp.float32),
    mesh=_MESH,
    scratch_types=[
        pltpu.VMEM((EPW // 16, 16), jnp.int32),
        pltpu.VMEM((NROW, 16), jnp.float32),
        pltpu.VMEM((NROW // 128, 128), jnp.int32),
        pltpu.VMEM_SHARED((NROW, 16), jnp.float32),
    ],
    compiler_params=_SC_FLAT,
)
def _deg_kernel(dst_hbm, out_hbm, dstbuf, partial, idrows, acc):
    c = lax.axis_index("c")
    s = lax.axis_index("s")
    wid = c * NS + s
    pltpu.sync_copy(dst_hbm.at[wid], dstbuf)

    zeros16 = jnp.zeros((16,), jnp.float32)

    @pl.loop(0, NROW)
    def _zero(r):
        partial[r, :] = zeros16

    iota16 = lax.broadcasted_iota(jnp.int32, (16,), 0)
    for k in range(NROW // 128):
        for j in range(8):
            idrows[k, pl.ds(j * 16, 16)] = iota16 + (k * 128 + j * 16)

    rows_per = NROW // NS
    pltpu.sync_copy(partial.at[pl.ds(s * rows_per, rows_per)],
                    acc.at[pl.ds(s * rows_per, rows_per)])
    plsc.subcore_barrier()

    @pl.loop(0, EPW // 16)
    def _count(i):
        n = dstbuf[i, :]
        row = jnp.right_shift(n, 4)
        col = jnp.bitwise_and(n, 15)
        cnt, last = plsc.scan_count(n)
        plsc.addupdate_scatter(partial, [row, col], cnt.astype(jnp.float32),
                               mask=last)

    for k in range(NROW // 128):
        pltpu.sync_copy(partial.at[pl.ds(k * 128, 128)],
                        acc.at[idrows.at[k]], add=True)
    plsc.subcore_barrier()

    @pl.when(s == 0)
    def _writeback():
        pltpu.sync_copy(acc, out_hbm.at[c])


@functools.partial(
    pl.kernel,
    out_type=jax.ShapeDtypeStruct((NC, N, D), jnp.float32),
    mesh=_MESH,
    scratch_types=(
        [
            pltpu.VMEM((NCHUNK, CH), jnp.int32),
            pltpu.VMEM((NCHUNK, CH), jnp.int32),
            pltpu.VMEM((1, TAIL), jnp.int32),
            pltpu.VMEM((1, TAIL), jnp.int32),
            pltpu.VMEM_SHARED((N, D), jnp.float32),
        ]
        + [pltpu.VMEM((CH, D), jnp.float32) for _ in range(NBUF)]
        + [pltpu.SemaphoreType.DMA for _ in range(NBUF)]
    ),
    compiler_params=_SC_FLAT,
)
def _agg_kernel(h_hbm, src_hbm, dst_hbm, stail_hbm, dtail_hbm, out_hbm,
                sidx, didx, stail, dtail, acc, *bufs_and_sems):
    rbufs = bufs_and_sems[:NBUF]
    gsems = bufs_and_sems[NBUF:]
    c = lax.axis_index("c")
    s = lax.axis_index("s")
    wid = c * NS + s
    pltpu.sync_copy(src_hbm.at[wid], sidx)
    pltpu.sync_copy(dst_hbm.at[wid], didx)
    pltpu.sync_copy(stail_hbm.at[wid], stail)
    pltpu.sync_copy(dtail_hbm.at[wid], dtail)

    for j in range(NBUF - 1):
        pltpu.async_copy(h_hbm.at[sidx.at[j]], rbufs[j], gsems[j])

    zeros16 = jnp.zeros((16,), jnp.float32)
    zb = rbufs[NBUF - 1]

    @pl.loop(0, CH)
    def _zero(r):
        for j in range(D // 16):
            zb[r, pl.ds(j * 16, 16)] = zeros16

    base = s * RBASE
    for k in range(RSPAN // CH):
        pltpu.sync_copy(zb, acc.at[pl.ds(base + k * CH, CH)])
    if RSPAN % CH:
        pltpu.sync_copy(zb.at[pl.ds(0, RSPAN % CH)],
                        acc.at[pl.ds(base + (RSPAN // CH) * CH, RSPAN % CH)])
    pltpu.async_copy(h_hbm.at[sidx.at[NBUF - 1]], zb, gsems[NBUF - 1])
    plsc.subcore_barrier()

    @pl.loop(0, NCHUNK // NBUF - 1)
    def _edges(k):
        i0 = k * NBUF
        for j in range(NBUF):
            pltpu.make_async_copy(h_hbm.at[sidx.at[i0 + j]],
                                  rbufs[j], gsems[j]).wait()
            pltpu.sync_copy(rbufs[j], acc.at[didx.at[i0 + j]], add=True)
            pltpu.async_copy(h_hbm.at[sidx.at[i0 + NBUF + j]], rbufs[j],
                             gsems[j])

    i0 = NCHUNK - NBUF
    pltpu.make_async_copy(h_hbm.at[sidx.at[i0]], rbufs[0], gsems[0]).wait()
    pltpu.sync_copy(rbufs[0], acc.at[didx.at[i0]], add=True)
    pltpu.async_copy(h_hbm.at[stail.at[0]], rbufs[0].at[pl.ds(0, TAIL)],
                     gsems[0])
    for j in range(1, NBUF):
        pltpu.make_async_copy(h_hbm.at[sidx.at[i0 + j]],
                              rbufs[j], gsems[j]).wait()
        pltpu.sync_copy(rbufs[j], acc.at[didx.at[i0 + j]], add=True)
    pltpu.make_async_copy(h_hbm.at[stail.at[0]],
                          rbufs[0].at[pl.ds(0, TAIL)], gsems[0]).wait()
    pltpu.sync_copy(rbufs[0].at[pl.ds(0, TAIL)], acc.at[dtail.at[0]],
                    add=True)

    plsc.subcore_barrier()
    pltpu.sync_copy(acc.at[pl.ds(base, RSPAN)],
                    out_hbm.at[c].at[pl.ds(base, RSPAN)])


def _mm_scale_body(x_ref, w_ref, deg_ref, hs_ref, dis_ref):
    dis = lax.rsqrt(deg_ref[...] + 1.0)
    dis_ref[...] = dis
    h = jnp.dot(x_ref[...], w_ref[...], preferred_element_type=jnp.float32)
    hs_ref[...] = h * dis


def _comb_body(p_ref, hs_ref, dis_ref, b_ref, g_ref, be_ref, w_ref,
               o_ref):
    t = (p_ref[0] + p_ref[1] + hs_ref[...]) * dis_ref[...] + b_ref[...]
    mean = jnp.mean(t, axis=0, keepdims=True)
    var = jnp.mean((t - mean) ** 2, axis=0, keepdims=True)
    bn = (t - mean) * lax.rsqrt(var + 1e-5) * g_ref[...] + be_ref[...]
    r = jnp.maximum(bn, 0.0)
    h2 = jnp.dot(r, w_ref[...], preferred_element_type=jnp.float32)
    o_ref[...] = h2 * dis_ref[...]


def _fin_body(p_ref, hs_ref, dis_ref, b_ref, o_ref):
    o_ref[...] = ((p_ref[0] + p_ref[1] + hs_ref[...]) * dis_ref[...]
                  + b_ref[...])


_f32 = jnp.float32
_mm_scale = pl.pallas_call(
    _mm_scale_body,
    out_shape=(jax.ShapeDtypeStruct((N, D), _f32),
               jax.ShapeDtypeStruct((N, 1), _f32)))
_comb = pl.pallas_call(_comb_body, out_shape=jax.ShapeDtypeStruct((N, D), _f32))
_fin = pl.pallas_call(_fin_body, out_shape=jax.ShapeDtypeStruct((N, D), _f32))


def kernel(x, edge_index, W1, b1, gamma1, beta1, W2, b2):
    src_w = edge_index[0].reshape(NW, EPW)
    dst_w2 = edge_index[1].reshape(NW, EPW)
    src_c = src_w[:, :NCHUNK * CH].reshape(NW, NCHUNK, CH)
    dst_c = dst_w2[:, :NCHUNK * CH].reshape(NW, NCHUNK, CH)
    src_t = src_w[:, NCHUNK * CH:].reshape(NW, 1, TAIL)
    dst_t = dst_w2[:, NCHUNK * CH:].reshape(NW, 1, TAIL)
    dst_w = edge_index[1].reshape(NW, EPW // 16, 16)

    degp = _deg_kernel(dst_w)
    deg2 = (degp[0] + degp[1]).reshape(-1)[:N].reshape(N, 1)

    h1s, dis = _mm_scale(x, W1, deg2)
    p = _agg_kernel(h1s, src_c, dst_c, src_t, dst_t)
    h2s = _comb(p, h1s, dis, b1.reshape(1, D), gamma1.reshape(1, D),
                beta1.reshape(1, D), W2)
    p2 = _agg_kernel(h2s, src_c, dst_c, src_t, dst_t)
    return _fin(p2, h2s, dis, b2.reshape(1, D))
